# Initial kernel scaffold; baseline (speedup 1.0000x reference)
import jax
import jax.numpy as jnp
from jax.experimental import pallas as pl


def kernel(inp, w):
    raise NotImplementedError("write your pallas kernel here")


# speedup vs baseline: 3.3919x; 3.3919x over previous
import functools

import jax
import jax.numpy as jnp
from jax import lax
from jax.experimental import pallas as pl
from jax.experimental.pallas import tpu as pltpu
from jax.experimental.pallas import tpu_sc as plsc

NUM_EMB = 100000
D = 128
B = 16384 * 26
NC = 2
NS = 16
NW = NC * NS
BPW = B // NW
CH = 128
NCH = BPW // CH
NBUF = 4

assert B == NW * NCH * CH


def _body(table_hbm, idx_hbm, out_hbm, idx_v, rows_v, gsems, ssem):
    wid = lax.axis_index("s") * NC + lax.axis_index("c")
    base = wid * BPW

    pltpu.sync_copy(idx_hbm.at[pl.ds(wid * NCH, NCH)], idx_v)

    def issue_gather(g, b):
        return pltpu.async_copy(table_hbm.at[idx_v.at[g]], rows_v.at[b], gsems[b])

    def wait_gather(g, b):
        pltpu.make_async_copy(table_hbm.at[idx_v.at[g]], rows_v.at[b], gsems[b]).wait()

    def issue_scatter(g, b):
        pltpu.async_copy(rows_v.at[b], out_hbm.at[pl.ds(base + g * CH, CH)], ssem)

    def wait_scatter_unit():
        pltpu.make_async_copy(rows_v.at[0], out_hbm.at[pl.ds(base, CH)], ssem).wait()

    for h in range(NBUF - 1):
        issue_gather(h, h)

    @pl.loop(0, NCH, step=NBUF)
    def chunk_group(g0):
        for b in range(NBUF):
            g = g0 + b
            wait_gather(g, b)
            issue_scatter(g, b)
            h = g + NBUF - 1
            hb = (b + NBUF - 1) % NBUF

            if b == 0:
                @pl.when(g >= 1)
                def _():
                    wait_scatter_unit()

                @pl.when(h < NCH)
                def _():
                    issue_gather(h, hb)
            else:
                @pl.when(h < NCH)
                def _():
                    wait_scatter_unit()
                    issue_gather(h, hb)

    for _ in range(NBUF - 1):
        wait_scatter_unit()


@jax.jit
def _embedding_gather(w, idx2d):
    mesh = plsc.VectorSubcoreMesh(core_axis_name="c", subcore_axis_name="s")
    f = functools.partial(
        pl.kernel,
        out_type=jax.ShapeDtypeStruct((B, D), jnp.float32),
        mesh=mesh,
        scratch_types=[
            pltpu.VMEM((NCH, CH), jnp.int32),
            pltpu.VMEM((NBUF, CH, D), jnp.float32),
            [pltpu.SemaphoreType.DMA] * NBUF,
            pltpu.SemaphoreType.DMA,
        ],
    )(_body)
    return f(w, idx2d)


def kernel(inp, w):
    idx2d = inp.reshape(NW * NCH, CH).astype(jnp.int32)
    out = _embedding_gather(w, idx2d)
    return out.reshape(inp.shape[0], inp.shape[1], D)
